# Initial kernel scaffold; baseline (speedup 1.0000x reference)
import jax
import jax.numpy as jnp
from jax.experimental import pallas as pl


def kernel(data, edge_index, W_rel, b_rel, W_root, Ws_rel, bs_rel, Ws_root, W1, b1, W2, b2, W3, b3):
    raise NotImplementedError("write your pallas kernel here")


# speedup vs baseline: 1.4413x; 1.4413x over previous
import functools

import jax
import jax.numpy as jnp
from jax import lax
from jax.experimental import pallas as pl
from jax.experimental.pallas import tpu as pltpu
from jax.experimental.pallas import tpu_sc as plsc

N = 10000
F = 128
H = 16
B = 2
E = 160000
K = 4000

NT = 16
ROWS_T = 624

NV = N // 16
KV = K // 16
TR = 256


def _sc_mesh():
    return plsc.VectorSubcoreMesh(core_axis_name="c", subcore_axis_name="s")


def _segsum(table, src_glob, dst_loc, zeros_tile):
    W = table.shape[1]
    CHUNK = 200
    NCH = E // CHUNK
    MAXJ = NCH // NT

    @functools.partial(
        pl.kernel,
        out_type=jax.ShapeDtypeStruct((B * N, W), jnp.float32),
        mesh=_sc_mesh(),
        scratch_types=[
            pltpu.VMEM((CHUNK,), jnp.int32),
            pltpu.VMEM((CHUNK,), jnp.int32),
            pltpu.VMEM((CHUNK, W), jnp.float32),
            pltpu.SemaphoreType.DMA,
            pltpu.VMEM_SHARED((N, W), jnp.float32),
        ],
    )
    def seg(table_hbm, src_hbm, dst_hbm, zeros_hbm, out_hbm,
            src_v, dst_v, rows_v, sem, acc):
        cid = lax.axis_index("c")
        sid = lax.axis_index("s")
        pltpu.sync_copy(zeros_hbm, acc.at[pl.ds(sid * ROWS_T, ROWS_T)])

        @pl.when(sid == NT - 1)
        def _():
            pltpu.sync_copy(zeros_hbm.at[pl.ds(0, 16)],
                            acc.at[pl.ds(NT * ROWS_T, 16)])

        plsc.subcore_barrier()
        ebase = cid * E

        for r in range(MAXJ):
            j = r * NT + sid
            base = ebase + j * CHUNK
            pltpu.sync_copy(src_hbm.at[pl.ds(base, CHUNK)], src_v)
            pltpu.sync_copy(dst_hbm.at[pl.ds(base, CHUNK)], dst_v)
            pltpu.async_copy(table_hbm.at[src_v], rows_v, sem).wait()
            for k in range(NT):
                @pl.when(sid == k)
                def _():
                    pltpu.sync_copy(rows_v, acc.at[dst_v], add=True)

                plsc.subcore_barrier()
        pltpu.sync_copy(
            acc.at[pl.ds(sid * ROWS_T, ROWS_T)],
            out_hbm.at[pl.ds(cid * N + sid * ROWS_T, ROWS_T)],
        )

        @pl.when(sid == NT - 1)
        def _():
            pltpu.sync_copy(
                acc.at[pl.ds(NT * ROWS_T, 16)],
                out_hbm.at[pl.ds(cid * N + NT * ROWS_T, 16)],
            )

    return seg(table, src_glob, dst_loc, zeros_tile)


def _topk_pool(keys, vals, h):
    @functools.partial(
        pl.kernel,
        out_type=jax.ShapeDtypeStruct((B, K, F), jnp.float32),
        mesh=_sc_mesh(),
        scratch_types=[
            pltpu.VMEM((N,), jnp.int32),
            pltpu.VMEM((N,), jnp.int32),
            pltpu.VMEM((N,), jnp.int32),
            pltpu.VMEM((N,), jnp.int32),
            pltpu.VMEM((256,), jnp.int32),
            pltpu.VMEM((256,), jnp.int32),
            pltpu.VMEM((TR,), jnp.float32),
            pltpu.VMEM((TR, F), jnp.float32),
            pltpu.SemaphoreType.DMA,
        ],
        compiler_params=pltpu.CompilerParams(needs_layout_passes=False),
    )
    def srt(keys_hbm, vals_hbm, h_hbm, hp_hbm,
            ka, va, kb, vb, hist, offs, sval_v, prow_v, sem):
        cid = lax.axis_index("c")
        sid = lax.axis_index("s")

        pltpu.sync_copy(keys_hbm.at[pl.ds(cid * N, N)], ka)
        pltpu.sync_copy(vals_hbm.at[pl.ds(cid * N, N)], va)
        for p in range(4):
            ink, inv = (ka, va) if p % 2 == 0 else (kb, vb)
            outk, outv = (kb, vb) if p % 2 == 0 else (ka, va)
            shift = 8 * p
            for t in range(16):
                hist[pl.ds(t * 16, 16)] = jnp.zeros((16,), jnp.int32)

            def hist_body(v, carry, ink=ink, shift=shift):
                kv = ink[pl.ds(v * 16, 16)]
                d = lax.shift_right_logical(kv, shift) & 0xFF
                c, last = plsc.scan_count(d)
                plsc.addupdate_scatter(hist, [d], c, mask=last)
                return carry

            lax.fori_loop(0, NV, hist_body, 0)
            carry = jnp.int32(0)
            for t in range(16):
                hv = hist[pl.ds(t * 16, 16)]
                cs = plsc.cumsum(hv)
                offs[pl.ds(t * 16, 16)] = cs - hv + carry
                carry = carry + cs[15]

            def perm_body(v, carry, ink=ink, inv=inv, outk=outk,
                          outv=outv, shift=shift):
                kv = ink[pl.ds(v * 16, 16)]
                vv = inv[pl.ds(v * 16, 16)]
                d = lax.shift_right_logical(kv, shift) & 0xFF
                c, last = plsc.scan_count(d)
                base = plsc.load_gather(offs, [d])
                pos = base + c - 1
                plsc.store_scatter(outk, [pos], kv)
                plsc.store_scatter(outv, [pos], vv)
                plsc.addupdate_scatter(offs, [d], c, mask=last)
                return carry

            lax.fori_loop(0, NV, perm_body, 0)

        rbase = sid * TR

        def score_body(t, carry):
            kk = ka[pl.ds(rbase + t * 16, 16)]
            u = kk ^ (jnp.int32(0x7FFFFFFF) & ~(kk >> 31))
            sval_v[pl.ds(t * 16, 16)] = plsc.bitcast(u, jnp.float32)
            return carry

        lax.fori_loop(0, TR // 16, score_body, 0)

        pltpu.async_copy(h_hbm.at[va.at[pl.ds(rbase, TR)]], prow_v, sem).wait()

        def scale_body(g, carry):
            s16 = sval_v[pl.ds(g * 16, 16)]
            ridx = g * 16 + lax.iota(jnp.int32, 16)
            for c in range(H):
                cidx = jnp.full((16,), c, jnp.int32)
                rv = plsc.load_gather(prow_v, [ridx, cidx])
                plsc.store_scatter(prow_v, [ridx, cidx], rv * s16)
            return carry

        lax.fori_loop(0, TR // 16, scale_body, 0)

        @pl.when(sid < NT - 1)
        def _out_full():
            pltpu.sync_copy(prow_v, hp_hbm.at[cid, pl.ds(rbase, TR)])

        @pl.when(sid == NT - 1)
        def _out_tail():
            pltpu.sync_copy(prow_v.at[pl.ds(0, K - (NT - 1) * TR)],
                            hp_hbm.at[cid, pl.ds(rbase, K - (NT - 1) * TR)])

    return srt(keys, vals, h)


def _fc_head(xf, w1, b1_2, w2, b2_2, w3, b3_2):
    CK = 6400
    steps = (K * H) // CK

    def body(xf_ref, w1_ref, b1_ref, w2_ref, b2_ref, w3_ref, b3_ref,
             out_ref, acc):
        i = pl.program_id(0)

        @pl.when(i == 0)
        def _():
            acc[...] = jnp.zeros_like(acc)

        acc[...] += lax.dot_general(xf_ref[...], w1_ref[...],
                                    (((1,), (1,)), ((), ())),
                                    preferred_element_type=jnp.float32)

        @pl.when(i == steps - 1)
        def _():
            y1 = jnp.maximum(acc[...] + b1_ref[...], 0.0)
            y2 = lax.dot_general(y1, w2_ref[...], (((1,), (1,)), ((), ())),
                                 preferred_element_type=jnp.float32)
            y2 = jnp.maximum(y2 + b2_ref[...], 0.0)
            y3 = lax.dot_general(y2, w3_ref[...], (((1,), (1,)), ((), ())),
                                 preferred_element_type=jnp.float32)
            out_ref[...] = y3 + b3_ref[0, 0]

    return pl.pallas_call(
        body,
        grid=(steps,),
        in_specs=[
            pl.BlockSpec((B, CK), lambda i: (0, i)),
            pl.BlockSpec((256, CK), lambda i: (0, i)),
            pl.BlockSpec((1, 256), lambda i: (0, 0)),
            pl.BlockSpec((64, 256), lambda i: (0, 0)),
            pl.BlockSpec((1, 64), lambda i: (0, 0)),
            pl.BlockSpec((8, 64), lambda i: (0, 0)),
            pl.BlockSpec((1, 1), lambda i: (0, 0)),
        ],
        out_specs=pl.BlockSpec((B, 8), lambda i: (0, 0)),
        out_shape=jax.ShapeDtypeStruct((B, 8), jnp.float32),
        scratch_shapes=[pltpu.VMEM((B, 256), jnp.float32)],
    )(xf, w1, b1_2, w2, b2_2, w3, b3_2)


def kernel(data, edge_index, W_rel, b_rel, W_root, Ws_rel, bs_rel, Ws_root,
           W1, b1, W2, b2, W3, b3):
    offs = jnp.arange(B, dtype=edge_index.dtype) * N
    src_glob = (edge_index[0][None, :] + offs[:, None]).reshape(-1)
    dst_loc = jnp.concatenate([edge_index[1], edge_index[1]])
    x = data[:, :N * F].reshape(B * N, F)

    zeros_f = jnp.zeros((ROWS_T, F), jnp.float32)

    agg = _segsum(x, src_glob, dst_loc, zeros_f)
    h = jax.nn.relu(agg @ W_rel.T + b_rel + x @ W_root.T)
    h_full = jnp.concatenate(
        [h, jnp.zeros((B * N, F - H), jnp.float32)], axis=1)
    agg2 = _segsum(h_full, src_glob, dst_loc, zeros_f)[:, :H]

    score = (agg2 @ Ws_rel.T + bs_rel + h @ Ws_root.T).reshape(B, N)
    score = jnp.tanh(score)
    u = lax.bitcast_convert_type(score.reshape(B * N), jnp.int32)
    keys = ~(u ^ ((u >> 31) | jnp.int32(-2147483648)))
    vals = jnp.arange(B * N, dtype=jnp.int32)

    hp = _topk_pool(keys, vals, h_full)[:, :, :H]

    w3p = jnp.concatenate([W3, jnp.zeros((7, 64), jnp.float32)], axis=0)
    out = _fc_head(hp.reshape(B, K * H), W1, b1.reshape(1, 256),
                   W2, b2.reshape(1, 64), w3p, b3.reshape(1, 1))
    return out[:, :1]


# speedup vs baseline: 2.3380x; 1.6222x over previous
import functools

import jax
import jax.numpy as jnp
from jax import lax
from jax.experimental import pallas as pl
from jax.experimental.pallas import tpu as pltpu
from jax.experimental.pallas import tpu_sc as plsc

N = 10000
F = 128
H = 16
B = 2
E = 160000
K = 4000

NT = 16
ROWS_T = 624

NV = N // 16
KV = K // 16
TR = 256


def _sc_mesh():
    return plsc.VectorSubcoreMesh(core_axis_name="c", subcore_axis_name="s")


def _segsum(table, src_glob, dst_loc, zeros_tile):
    W = table.shape[1]
    CHUNK = 200 if W == F else 1000
    NCH = E // CHUNK
    MAXJ = NCH // NT
    params = (None if W == F
              else pltpu.CompilerParams(use_tc_tiling_on_sc=False))

    @functools.partial(
        pl.kernel,
        out_type=jax.ShapeDtypeStruct((B * N, W), jnp.float32),
        mesh=_sc_mesh(),
        scratch_types=[
            pltpu.VMEM((CHUNK,), jnp.int32),
            pltpu.VMEM((CHUNK,), jnp.int32),
            pltpu.VMEM((CHUNK, W), jnp.float32),
            pltpu.SemaphoreType.DMA,
            pltpu.VMEM_SHARED((N, W), jnp.float32),
        ],
        compiler_params=params,
    )
    def seg(table_hbm, src_hbm, dst_hbm, zeros_hbm, out_hbm,
            src_v, dst_v, rows_v, sem, acc):
        cid = lax.axis_index("c")
        sid = lax.axis_index("s")
        pltpu.sync_copy(zeros_hbm, acc.at[pl.ds(sid * ROWS_T, ROWS_T)])

        @pl.when(sid == NT - 1)
        def _():
            pltpu.sync_copy(zeros_hbm.at[pl.ds(0, 16)],
                            acc.at[pl.ds(NT * ROWS_T, 16)])

        plsc.subcore_barrier()
        ebase = cid * E

        for r in range(MAXJ):
            j = r * NT + sid
            base = ebase + j * CHUNK
            pltpu.sync_copy(src_hbm.at[pl.ds(base, CHUNK)], src_v)
            pltpu.sync_copy(dst_hbm.at[pl.ds(base, CHUNK)], dst_v)
            pltpu.async_copy(table_hbm.at[src_v], rows_v, sem).wait()
            for k in range(NT):
                @pl.when(sid == k)
                def _():
                    pltpu.sync_copy(rows_v, acc.at[dst_v], add=True)

                plsc.subcore_barrier()
        pltpu.sync_copy(
            acc.at[pl.ds(sid * ROWS_T, ROWS_T)],
            out_hbm.at[pl.ds(cid * N + sid * ROWS_T, ROWS_T)],
        )

        @pl.when(sid == NT - 1)
        def _():
            pltpu.sync_copy(
                acc.at[pl.ds(NT * ROWS_T, 16)],
                out_hbm.at[pl.ds(cid * N + NT * ROWS_T, 16)],
            )

    return seg(table, src_glob, dst_loc, zeros_tile)


def _topk_pool(keys, vals, h):
    @functools.partial(
        pl.kernel,
        out_type=jax.ShapeDtypeStruct((B, K, H), jnp.float32),
        mesh=_sc_mesh(),
        scratch_types=[
            pltpu.VMEM((N,), jnp.int32),
            pltpu.VMEM((N,), jnp.int32),
            pltpu.VMEM((N,), jnp.int32),
            pltpu.VMEM((N,), jnp.int32),
            pltpu.VMEM((256,), jnp.int32),
            pltpu.VMEM((256,), jnp.int32),
            pltpu.VMEM((TR,), jnp.float32),
            pltpu.VMEM((TR, H), jnp.float32),
            pltpu.SemaphoreType.DMA,
        ],
        compiler_params=pltpu.CompilerParams(needs_layout_passes=False,
                                             use_tc_tiling_on_sc=False),
    )
    def srt(keys_hbm, vals_hbm, h_hbm, hp_hbm,
            ka, va, kb, vb, hist, offs, sval_v, prow_v, sem):
        cid = lax.axis_index("c")
        sid = lax.axis_index("s")

        pltpu.sync_copy(keys_hbm.at[pl.ds(cid * N, N)], ka)
        pltpu.sync_copy(vals_hbm.at[pl.ds(cid * N, N)], va)
        for p in range(4):
            ink, inv = (ka, va) if p % 2 == 0 else (kb, vb)
            outk, outv = (kb, vb) if p % 2 == 0 else (ka, va)
            shift = 8 * p
            for t in range(16):
                hist[pl.ds(t * 16, 16)] = jnp.zeros((16,), jnp.int32)

            def hist_body(v, carry, ink=ink, shift=shift):
                kv = ink[pl.ds(v * 16, 16)]
                d = lax.shift_right_logical(kv, shift) & 0xFF
                c, last = plsc.scan_count(d)
                plsc.addupdate_scatter(hist, [d], c, mask=last)
                return carry

            lax.fori_loop(0, NV, hist_body, 0)
            carry = jnp.int32(0)
            for t in range(16):
                hv = hist[pl.ds(t * 16, 16)]
                cs = plsc.cumsum(hv)
                offs[pl.ds(t * 16, 16)] = cs - hv + carry
                carry = carry + cs[15]

            def perm_body(v, carry, ink=ink, inv=inv, outk=outk,
                          outv=outv, shift=shift):
                kv = ink[pl.ds(v * 16, 16)]
                vv = inv[pl.ds(v * 16, 16)]
                d = lax.shift_right_logical(kv, shift) & 0xFF
                c, last = plsc.scan_count(d)
                base = plsc.load_gather(offs, [d])
                pos = base + c - 1
                plsc.store_scatter(outk, [pos], kv)
                plsc.store_scatter(outv, [pos], vv)
                plsc.addupdate_scatter(offs, [d], c, mask=last)
                return carry

            lax.fori_loop(0, NV, perm_body, 0)

        rbase = sid * TR

        def score_body(t, carry):
            kk = ka[pl.ds(rbase + t * 16, 16)]
            u = kk ^ (jnp.int32(0x7FFFFFFF) & ~(kk >> 31))
            sval_v[pl.ds(t * 16, 16)] = plsc.bitcast(u, jnp.float32)
            return carry

        lax.fori_loop(0, TR // 16, score_body, 0)

        pltpu.async_copy(h_hbm.at[va.at[pl.ds(rbase, TR)]], prow_v, sem).wait()

        def scale_body(g, carry):
            s16 = sval_v[pl.ds(g * 16, 16)]
            ridx = g * 16 + lax.iota(jnp.int32, 16)
            for c in range(H):
                cidx = jnp.full((16,), c, jnp.int32)
                rv = plsc.load_gather(prow_v, [ridx, cidx])
                plsc.store_scatter(prow_v, [ridx, cidx], rv * s16)
            return carry

        lax.fori_loop(0, TR // 16, scale_body, 0)

        @pl.when(sid < NT - 1)
        def _out_full():
            pltpu.sync_copy(prow_v, hp_hbm.at[cid, pl.ds(rbase, TR)])

        @pl.when(sid == NT - 1)
        def _out_tail():
            pltpu.sync_copy(prow_v.at[pl.ds(0, K - (NT - 1) * TR)],
                            hp_hbm.at[cid, pl.ds(rbase, K - (NT - 1) * TR)])

    return srt(keys, vals, h)


def _fc_head(xf, w1, b1_2, w2, b2_2, w3, b3_2):
    CK = 6400
    steps = (K * H) // CK

    def body(xf_ref, w1_ref, b1_ref, w2_ref, b2_ref, w3_ref, b3_ref,
             out_ref, acc):
        i = pl.program_id(0)

        @pl.when(i == 0)
        def _():
            acc[...] = jnp.zeros_like(acc)

        acc[...] += lax.dot_general(xf_ref[...], w1_ref[...],
                                    (((1,), (1,)), ((), ())),
                                    preferred_element_type=jnp.float32)

        @pl.when(i == steps - 1)
        def _():
            y1 = jnp.maximum(acc[...] + b1_ref[...], 0.0)
            y2 = lax.dot_general(y1, w2_ref[...], (((1,), (1,)), ((), ())),
                                 preferred_element_type=jnp.float32)
            y2 = jnp.maximum(y2 + b2_ref[...], 0.0)
            y3 = lax.dot_general(y2, w3_ref[...], (((1,), (1,)), ((), ())),
                                 preferred_element_type=jnp.float32)
            out_ref[...] = y3 + b3_ref[0, 0]

    return pl.pallas_call(
        body,
        grid=(steps,),
        in_specs=[
            pl.BlockSpec((B, CK), lambda i: (0, i)),
            pl.BlockSpec((256, CK), lambda i: (0, i)),
            pl.BlockSpec((1, 256), lambda i: (0, 0)),
            pl.BlockSpec((64, 256), lambda i: (0, 0)),
            pl.BlockSpec((1, 64), lambda i: (0, 0)),
            pl.BlockSpec((8, 64), lambda i: (0, 0)),
            pl.BlockSpec((1, 1), lambda i: (0, 0)),
        ],
        out_specs=pl.BlockSpec((B, 8), lambda i: (0, 0)),
        out_shape=jax.ShapeDtypeStruct((B, 8), jnp.float32),
        scratch_shapes=[pltpu.VMEM((B, 256), jnp.float32)],
    )(xf, w1, b1_2, w2, b2_2, w3, b3_2)


def kernel(data, edge_index, W_rel, b_rel, W_root, Ws_rel, bs_rel, Ws_root,
           W1, b1, W2, b2, W3, b3):
    offs = jnp.arange(B, dtype=edge_index.dtype) * N
    src_glob = (edge_index[0][None, :] + offs[:, None]).reshape(-1)
    dst_loc = jnp.concatenate([edge_index[1], edge_index[1]])
    x = data[:, :N * F].reshape(B * N, F)

    zeros_f = jnp.zeros((ROWS_T, F), jnp.float32)
    zeros_h = jnp.zeros((ROWS_T, H), jnp.float32)

    agg = _segsum(x, src_glob, dst_loc, zeros_f)
    h = jax.nn.relu(agg @ W_rel.T + b_rel + x @ W_root.T)
    agg2 = _segsum(h, src_glob, dst_loc, zeros_h)

    score = (agg2 @ Ws_rel.T + bs_rel + h @ Ws_root.T).reshape(B, N)
    score = jnp.tanh(score)
    u = lax.bitcast_convert_type(score.reshape(B * N), jnp.int32)
    keys = ~(u ^ ((u >> 31) | jnp.int32(-2147483648)))
    vals = jnp.arange(B * N, dtype=jnp.int32)

    hp = _topk_pool(keys, vals, h)

    w3p = jnp.concatenate([W3, jnp.zeros((7, 64), jnp.float32)], axis=0)
    out = _fc_head(hp.reshape(B, K * H), W1, b1.reshape(1, 256),
                   W2, b2.reshape(1, 64), w3p, b3.reshape(1, 1))
    return out[:, :1]


# speedup vs baseline: 2.4331x; 1.0407x over previous
import functools

import jax
import jax.numpy as jnp
from jax import lax
from jax.experimental import pallas as pl
from jax.experimental.pallas import tpu as pltpu
from jax.experimental.pallas import tpu_sc as plsc

N = 10000
F = 128
H = 16
B = 2
E = 160000
K = 4000

NT = 16
ROWS_T = 624

NV = N // 16
KV = K // 16
TR = 256


def _sc_mesh():
    return plsc.VectorSubcoreMesh(core_axis_name="c", subcore_axis_name="s")


def _segsum(table, src_glob, dst_loc, zeros_tile):
    W = table.shape[1]
    CHUNK = 200 if W == F else 1000
    NCH = E // CHUNK
    MAXJ = NCH // NT
    params = (None if W == F
              else pltpu.CompilerParams(use_tc_tiling_on_sc=False))

    @functools.partial(
        pl.kernel,
        out_type=jax.ShapeDtypeStruct((B * N, W), jnp.float32),
        mesh=_sc_mesh(),
        scratch_types=[
            pltpu.VMEM((CHUNK,), jnp.int32),
            pltpu.VMEM((CHUNK,), jnp.int32),
            pltpu.VMEM((CHUNK, W), jnp.float32),
            pltpu.SemaphoreType.DMA,
            pltpu.VMEM_SHARED((N, W), jnp.float32),
        ],
        compiler_params=params,
    )
    def seg(table_hbm, src_hbm, dst_hbm, zeros_hbm, out_hbm,
            src_v, dst_v, rows_v, sem, acc):
        cid = lax.axis_index("c")
        sid = lax.axis_index("s")
        pltpu.sync_copy(zeros_hbm, acc.at[pl.ds(sid * ROWS_T, ROWS_T)])

        @pl.when(sid == NT - 1)
        def _():
            pltpu.sync_copy(zeros_hbm.at[pl.ds(0, 16)],
                            acc.at[pl.ds(NT * ROWS_T, 16)])

        plsc.subcore_barrier()
        ebase = cid * E

        for r in range(MAXJ):
            j = r * NT + sid
            base = ebase + j * CHUNK
            pltpu.sync_copy(src_hbm.at[pl.ds(base, CHUNK)], src_v)
            pltpu.sync_copy(dst_hbm.at[pl.ds(base, CHUNK)], dst_v)
            pltpu.async_copy(table_hbm.at[src_v], rows_v, sem).wait()
            for k in range(NT):
                @pl.when(sid == k)
                def _():
                    pltpu.sync_copy(rows_v, acc.at[dst_v], add=True)

                plsc.subcore_barrier()
        pltpu.sync_copy(
            acc.at[pl.ds(sid * ROWS_T, ROWS_T)],
            out_hbm.at[pl.ds(cid * N + sid * ROWS_T, ROWS_T)],
        )

        @pl.when(sid == NT - 1)
        def _():
            pltpu.sync_copy(
                acc.at[pl.ds(NT * ROWS_T, 16)],
                out_hbm.at[pl.ds(cid * N + NT * ROWS_T, 16)],
            )

    return seg(table, src_glob, dst_loc, zeros_tile)


def _topk_pool(keys, vals, h):
    @functools.partial(
        pl.kernel,
        out_type=jax.ShapeDtypeStruct((B, K, H), jnp.float32),
        mesh=_sc_mesh(),
        scratch_types=[
            pltpu.VMEM((N,), jnp.int32),
            pltpu.VMEM((N,), jnp.int32),
            pltpu.VMEM((N,), jnp.int32),
            pltpu.VMEM((N,), jnp.int32),
            pltpu.VMEM((256,), jnp.int32),
            pltpu.VMEM((256,), jnp.int32),
            pltpu.VMEM((TR,), jnp.float32),
            pltpu.VMEM((TR, H), jnp.float32),
            pltpu.SemaphoreType.DMA,
        ],
        compiler_params=pltpu.CompilerParams(needs_layout_passes=False,
                                             use_tc_tiling_on_sc=False),
    )
    def srt(keys_hbm, vals_hbm, h_hbm, hp_hbm,
            ka, va, kb, vb, hist, offs, sval_v, prow_v, sem):
        cid = lax.axis_index("c")
        sid = lax.axis_index("s")

        pltpu.sync_copy(keys_hbm.at[pl.ds(cid * N, N)], ka)
        pltpu.sync_copy(vals_hbm.at[pl.ds(cid * N, N)], va)
        for p in range(4):
            ink, inv = (ka, va) if p % 2 == 0 else (kb, vb)
            outk, outv = (kb, vb) if p % 2 == 0 else (ka, va)
            shift = 8 * p
            for t in range(16):
                hist[pl.ds(t * 16, 16)] = jnp.zeros((16,), jnp.int32)

            def hist_body(v, carry, ink=ink, shift=shift):
                kv = ink[pl.ds(v * 16, 16)]
                d = lax.shift_right_logical(kv, shift) & 0xFF
                c, last = plsc.scan_count(d)
                plsc.addupdate_scatter(hist, [d], c, mask=last)
                return carry

            lax.fori_loop(0, NV, hist_body, 0)
            carry = jnp.int32(0)
            for t in range(16):
                hv = hist[pl.ds(t * 16, 16)]
                cs = plsc.cumsum(hv)
                offs[pl.ds(t * 16, 16)] = cs - hv + carry
                carry = carry + cs[15]

            def perm_body(v, carry, ink=ink, inv=inv, outk=outk,
                          outv=outv, shift=shift):
                kv = ink[pl.ds(v * 16, 16)]
                vv = inv[pl.ds(v * 16, 16)]
                d = lax.shift_right_logical(kv, shift) & 0xFF
                c, last = plsc.scan_count(d)
                base = plsc.load_gather(offs, [d])
                pos = base + c - 1
                plsc.store_scatter(outk, [pos], kv)
                plsc.store_scatter(outv, [pos], vv)
                plsc.addupdate_scatter(offs, [d], c, mask=last)
                return carry

            lax.fori_loop(0, NV, perm_body, 0)

        rbase = sid * TR

        def score_body(t, carry):
            kk = ka[pl.ds(rbase + t * 16, 16)]
            u = kk ^ (jnp.int32(0x7FFFFFFF) & ~(kk >> 31))
            sval_v[pl.ds(t * 16, 16)] = plsc.bitcast(u, jnp.float32)
            return carry

        lax.fori_loop(0, TR // 16, score_body, 0)

        pltpu.async_copy(h_hbm.at[va.at[pl.ds(rbase, TR)]], prow_v, sem).wait()

        def scale_body(g, carry):
            s16 = sval_v[pl.ds(g * 16, 16)]
            ridx = g * 16 + lax.iota(jnp.int32, 16)
            for c in range(H):
                cidx = jnp.full((16,), c, jnp.int32)
                rv = plsc.load_gather(prow_v, [ridx, cidx])
                plsc.store_scatter(prow_v, [ridx, cidx], rv * s16)
            return carry

        lax.fori_loop(0, TR // 16, scale_body, 0)

        @pl.when(sid < NT - 1)
        def _out_full():
            pltpu.sync_copy(prow_v, hp_hbm.at[cid, pl.ds(rbase, TR)])

        @pl.when(sid == NT - 1)
        def _out_tail():
            pltpu.sync_copy(prow_v.at[pl.ds(0, K - (NT - 1) * TR)],
                            hp_hbm.at[cid, pl.ds(rbase, K - (NT - 1) * TR)])

    return srt(keys, vals, h)


def _fc_head(xf, w1, b1_2, w2, b2_2, w3, b3_2):
    CK = 6400
    steps = (K * H) // CK

    def body(xf_ref, w1_ref, b1_ref, w2_ref, b2_ref, w3_ref, b3_ref,
             out_ref, acc):
        i = pl.program_id(0)

        @pl.when(i == 0)
        def _():
            acc[...] = jnp.zeros_like(acc)

        acc[...] += lax.dot_general(xf_ref[...], w1_ref[...],
                                    (((1,), (1,)), ((), ())),
                                    preferred_element_type=jnp.float32)

        @pl.when(i == steps - 1)
        def _():
            y1 = jnp.maximum(acc[...] + b1_ref[...], 0.0)
            y2 = lax.dot_general(y1, w2_ref[...], (((1,), (1,)), ((), ())),
                                 preferred_element_type=jnp.float32)
            y2 = jnp.maximum(y2 + b2_ref[...], 0.0)
            y3 = lax.dot_general(y2, w3_ref[...], (((1,), (1,)), ((), ())),
                                 preferred_element_type=jnp.float32)
            out_ref[...] = y3 + b3_ref[0, 0]

    return pl.pallas_call(
        body,
        grid=(steps,),
        in_specs=[
            pl.BlockSpec((B, CK), lambda i: (0, i)),
            pl.BlockSpec((256, CK), lambda i: (0, i)),
            pl.BlockSpec((1, 256), lambda i: (0, 0)),
            pl.BlockSpec((64, 256), lambda i: (0, 0)),
            pl.BlockSpec((1, 64), lambda i: (0, 0)),
            pl.BlockSpec((8, 64), lambda i: (0, 0)),
            pl.BlockSpec((1, 1), lambda i: (0, 0)),
        ],
        out_specs=pl.BlockSpec((B, 8), lambda i: (0, 0)),
        out_shape=jax.ShapeDtypeStruct((B, 8), jnp.float32),
        scratch_shapes=[pltpu.VMEM((B, 256), jnp.float32)],
    )(xf, w1, b1_2, w2, b2_2, w3, b3_2)


def kernel(data, edge_index, W_rel, b_rel, W_root, Ws_rel, bs_rel, Ws_root,
           W1, b1, W2, b2, W3, b3):
    offs = jnp.arange(B, dtype=edge_index.dtype) * N
    src_glob = (edge_index[0][None, :] + offs[:, None]).reshape(-1)
    dst_loc = jnp.concatenate([edge_index[1], edge_index[1]])
    x = data[:, :N * F].reshape(B * N, F)

    zeros_64 = jnp.zeros((ROWS_T, F // 2), jnp.float32)
    zeros_h = jnp.zeros((ROWS_T, H), jnp.float32)

    agg = jnp.concatenate(
        [_segsum(x[:, :F // 2], src_glob, dst_loc, zeros_64),
         _segsum(x[:, F // 2:], src_glob, dst_loc, zeros_64)], axis=1)
    h = jax.nn.relu(agg @ W_rel.T + b_rel + x @ W_root.T)
    agg2 = _segsum(h, src_glob, dst_loc, zeros_h)

    score = (agg2 @ Ws_rel.T + bs_rel + h @ Ws_root.T).reshape(B, N)
    score = jnp.tanh(score)
    u = lax.bitcast_convert_type(score.reshape(B * N), jnp.int32)
    keys = ~(u ^ ((u >> 31) | jnp.int32(-2147483648)))
    vals = jnp.arange(B * N, dtype=jnp.int32)

    hp = _topk_pool(keys, vals, h)

    w3p = jnp.concatenate([W3, jnp.zeros((7, 64), jnp.float32)], axis=0)
    out = _fc_head(hp.reshape(B, K * H), W1, b1.reshape(1, 256),
                   W2, b2.reshape(1, 64), w3p, b3.reshape(1, 1))
    return out[:, :1]


# speedup vs baseline: 2.4503x; 1.0070x over previous
import functools

import jax
import jax.numpy as jnp
from jax import lax
from jax.experimental import pallas as pl
from jax.experimental.pallas import tpu as pltpu
from jax.experimental.pallas import tpu_sc as plsc

N = 10000
F = 128
H = 16
B = 2
E = 160000
K = 4000

NT = 16
ROWS_T = 624

NV = N // 16
KV = K // 16
TR = 256


def _sc_mesh():
    return plsc.VectorSubcoreMesh(core_axis_name="c", subcore_axis_name="s")


def _segsum(table, src_glob, dst_loc, zeros_tile):
    W = table.shape[1]
    CHUNK = 2000
    NCH = E // CHUNK
    MAXJ = NCH // NT
    params = (None if W == F
              else pltpu.CompilerParams(use_tc_tiling_on_sc=False))

    @functools.partial(
        pl.kernel,
        out_type=jax.ShapeDtypeStruct((B * N, W), jnp.float32),
        mesh=_sc_mesh(),
        scratch_types=[
            pltpu.VMEM((CHUNK,), jnp.int32),
            pltpu.VMEM((CHUNK,), jnp.int32),
            pltpu.VMEM((CHUNK, W), jnp.float32),
            pltpu.SemaphoreType.DMA,
            pltpu.VMEM_SHARED((N, W), jnp.float32),
        ],
        compiler_params=params,
    )
    def seg(table_hbm, src_hbm, dst_hbm, zeros_hbm, out_hbm,
            src_v, dst_v, rows_v, sem, acc):
        cid = lax.axis_index("c")
        sid = lax.axis_index("s")
        pltpu.sync_copy(zeros_hbm, acc.at[pl.ds(sid * ROWS_T, ROWS_T)])

        @pl.when(sid == NT - 1)
        def _():
            pltpu.sync_copy(zeros_hbm.at[pl.ds(0, 16)],
                            acc.at[pl.ds(NT * ROWS_T, 16)])

        plsc.subcore_barrier()
        ebase = cid * E

        for r in range(MAXJ):
            j = r * NT + sid
            base = ebase + j * CHUNK
            pltpu.sync_copy(src_hbm.at[pl.ds(base, CHUNK)], src_v)
            pltpu.sync_copy(dst_hbm.at[pl.ds(base, CHUNK)], dst_v)
            pltpu.async_copy(table_hbm.at[src_v], rows_v, sem).wait()
            for k in range(NT):
                @pl.when(sid == k)
                def _():
                    pltpu.sync_copy(rows_v, acc.at[dst_v], add=True)

                plsc.subcore_barrier()
        pltpu.sync_copy(
            acc.at[pl.ds(sid * ROWS_T, ROWS_T)],
            out_hbm.at[pl.ds(cid * N + sid * ROWS_T, ROWS_T)],
        )

        @pl.when(sid == NT - 1)
        def _():
            pltpu.sync_copy(
                acc.at[pl.ds(NT * ROWS_T, 16)],
                out_hbm.at[pl.ds(cid * N + NT * ROWS_T, 16)],
            )

    return seg(table, src_glob, dst_loc, zeros_tile)


def _topk_pool(keys, vals, h):
    @functools.partial(
        pl.kernel,
        out_type=jax.ShapeDtypeStruct((B, K, H), jnp.float32),
        mesh=_sc_mesh(),
        scratch_types=[
            pltpu.VMEM((N,), jnp.int32),
            pltpu.VMEM((N,), jnp.int32),
            pltpu.VMEM((N,), jnp.int32),
            pltpu.VMEM((N,), jnp.int32),
            pltpu.VMEM((256,), jnp.int32),
            pltpu.VMEM((256,), jnp.int32),
            pltpu.VMEM((TR,), jnp.float32),
            pltpu.VMEM((TR, H), jnp.float32),
            pltpu.SemaphoreType.DMA,
        ],
        compiler_params=pltpu.CompilerParams(needs_layout_passes=False,
                                             use_tc_tiling_on_sc=False),
    )
    def srt(keys_hbm, vals_hbm, h_hbm, hp_hbm,
            ka, va, kb, vb, hist, offs, sval_v, prow_v, sem):
        cid = lax.axis_index("c")
        sid = lax.axis_index("s")

        pltpu.sync_copy(keys_hbm.at[pl.ds(cid * N, N)], ka)
        pltpu.sync_copy(vals_hbm.at[pl.ds(cid * N, N)], va)
        for p in range(4):
            ink, inv = (ka, va) if p % 2 == 0 else (kb, vb)
            outk, outv = (kb, vb) if p % 2 == 0 else (ka, va)
            shift = 8 * p
            for t in range(16):
                hist[pl.ds(t * 16, 16)] = jnp.zeros((16,), jnp.int32)

            def hist_body(v, carry, ink=ink, shift=shift):
                kv = ink[pl.ds(v * 16, 16)]
                d = lax.shift_right_logical(kv, shift) & 0xFF
                c, last = plsc.scan_count(d)
                plsc.addupdate_scatter(hist, [d], c, mask=last)
                return carry

            lax.fori_loop(0, NV, hist_body, 0)
            carry = jnp.int32(0)
            for t in range(16):
                hv = hist[pl.ds(t * 16, 16)]
                cs = plsc.cumsum(hv)
                offs[pl.ds(t * 16, 16)] = cs - hv + carry
                carry = carry + cs[15]

            def perm_body(v, carry, ink=ink, inv=inv, outk=outk,
                          outv=outv, shift=shift):
                kv = ink[pl.ds(v * 16, 16)]
                vv = inv[pl.ds(v * 16, 16)]
                d = lax.shift_right_logical(kv, shift) & 0xFF
                c, last = plsc.scan_count(d)
                base = plsc.load_gather(offs, [d])
                pos = base + c - 1
                plsc.store_scatter(outk, [pos], kv)
                plsc.store_scatter(outv, [pos], vv)
                plsc.addupdate_scatter(offs, [d], c, mask=last)
                return carry

            lax.fori_loop(0, NV, perm_body, 0)

        rbase = sid * TR

        def score_body(t, carry):
            kk = ka[pl.ds(rbase + t * 16, 16)]
            u = kk ^ (jnp.int32(0x7FFFFFFF) & ~(kk >> 31))
            sval_v[pl.ds(t * 16, 16)] = plsc.bitcast(u, jnp.float32)
            return carry

        lax.fori_loop(0, TR // 16, score_body, 0)

        pltpu.async_copy(h_hbm.at[va.at[pl.ds(rbase, TR)]], prow_v, sem).wait()

        def scale_body(g, carry):
            s16 = sval_v[pl.ds(g * 16, 16)]
            ridx = g * 16 + lax.iota(jnp.int32, 16)
            for c in range(H):
                cidx = jnp.full((16,), c, jnp.int32)
                rv = plsc.load_gather(prow_v, [ridx, cidx])
                plsc.store_scatter(prow_v, [ridx, cidx], rv * s16)
            return carry

        lax.fori_loop(0, TR // 16, scale_body, 0)

        @pl.when(sid < NT - 1)
        def _out_full():
            pltpu.sync_copy(prow_v, hp_hbm.at[cid, pl.ds(rbase, TR)])

        @pl.when(sid == NT - 1)
        def _out_tail():
            pltpu.sync_copy(prow_v.at[pl.ds(0, K - (NT - 1) * TR)],
                            hp_hbm.at[cid, pl.ds(rbase, K - (NT - 1) * TR)])

    return srt(keys, vals, h)


def _segsum_2x64(xlo, xhi, src_glob, dst_loc, zeros_tile):
    W = F // 2
    CHUNK = 1000
    NCH = E // CHUNK
    MAXJ = NCH // NT

    @functools.partial(
        pl.kernel,
        out_type=(jax.ShapeDtypeStruct((B * N, W), jnp.float32),
                  jax.ShapeDtypeStruct((B * N, W), jnp.float32)),
        mesh=_sc_mesh(),
        scratch_types=[
            pltpu.VMEM((CHUNK,), jnp.int32),
            pltpu.VMEM((CHUNK,), jnp.int32),
            pltpu.VMEM((CHUNK, W), jnp.float32),
            pltpu.SemaphoreType.DMA,
            pltpu.VMEM_SHARED((N, W), jnp.float32),
        ],
        compiler_params=pltpu.CompilerParams(use_tc_tiling_on_sc=False),
    )
    def seg(xlo_hbm, xhi_hbm, src_hbm, dst_hbm, zeros_hbm,
            outlo_hbm, outhi_hbm, src_v, dst_v, rows_v, sem, acc):
        cid = lax.axis_index("c")
        sid = lax.axis_index("s")
        ebase = cid * E
        for table_hbm, out_hbm in ((xlo_hbm, outlo_hbm), (xhi_hbm, outhi_hbm)):
            pltpu.sync_copy(zeros_hbm, acc.at[pl.ds(sid * ROWS_T, ROWS_T)])

            @pl.when(sid == NT - 1)
            def _():
                pltpu.sync_copy(zeros_hbm.at[pl.ds(0, 16)],
                                acc.at[pl.ds(NT * ROWS_T, 16)])

            plsc.subcore_barrier()
            for r in range(MAXJ):
                j = r * NT + sid
                base = ebase + j * CHUNK
                pltpu.sync_copy(src_hbm.at[pl.ds(base, CHUNK)], src_v)
                pltpu.sync_copy(dst_hbm.at[pl.ds(base, CHUNK)], dst_v)
                pltpu.async_copy(table_hbm.at[src_v], rows_v, sem).wait()
                for k in range(NT):
                    @pl.when(sid == k)
                    def _():
                        pltpu.sync_copy(rows_v, acc.at[dst_v], add=True)

                    plsc.subcore_barrier()

            pltpu.sync_copy(
                acc.at[pl.ds(sid * ROWS_T, ROWS_T)],
                out_hbm.at[pl.ds(cid * N + sid * ROWS_T, ROWS_T)],
            )

            @pl.when(sid == NT - 1)
            def _():
                pltpu.sync_copy(
                    acc.at[pl.ds(NT * ROWS_T, 16)],
                    out_hbm.at[pl.ds(cid * N + NT * ROWS_T, 16)],
                )

    return seg(xlo, xhi, src_glob, dst_loc, zeros_tile)


def _fc_head(xf, w1, b1_2, w2, b2_2, w3, b3_2):
    CK = 6400
    steps = (K * H) // CK

    def body(xf_ref, w1_ref, b1_ref, w2_ref, b2_ref, w3_ref, b3_ref,
             out_ref, acc):
        i = pl.program_id(0)

        @pl.when(i == 0)
        def _():
            acc[...] = jnp.zeros_like(acc)

        acc[...] += lax.dot_general(xf_ref[...], w1_ref[...],
                                    (((1,), (1,)), ((), ())),
                                    preferred_element_type=jnp.float32)

        @pl.when(i == steps - 1)
        def _():
            y1 = jnp.maximum(acc[...] + b1_ref[...], 0.0)
            y2 = lax.dot_general(y1, w2_ref[...], (((1,), (1,)), ((), ())),
                                 preferred_element_type=jnp.float32)
            y2 = jnp.maximum(y2 + b2_ref[...], 0.0)
            y3 = lax.dot_general(y2, w3_ref[...], (((1,), (1,)), ((), ())),
                                 preferred_element_type=jnp.float32)
            out_ref[...] = y3 + b3_ref[0, 0]

    return pl.pallas_call(
        body,
        grid=(steps,),
        in_specs=[
            pl.BlockSpec((B, CK), lambda i: (0, i)),
            pl.BlockSpec((256, CK), lambda i: (0, i)),
            pl.BlockSpec((1, 256), lambda i: (0, 0)),
            pl.BlockSpec((64, 256), lambda i: (0, 0)),
            pl.BlockSpec((1, 64), lambda i: (0, 0)),
            pl.BlockSpec((8, 64), lambda i: (0, 0)),
            pl.BlockSpec((1, 1), lambda i: (0, 0)),
        ],
        out_specs=pl.BlockSpec((B, 8), lambda i: (0, 0)),
        out_shape=jax.ShapeDtypeStruct((B, 8), jnp.float32),
        scratch_shapes=[pltpu.VMEM((B, 256), jnp.float32)],
    )(xf, w1, b1_2, w2, b2_2, w3, b3_2)


def kernel(data, edge_index, W_rel, b_rel, W_root, Ws_rel, bs_rel, Ws_root,
           W1, b1, W2, b2, W3, b3):
    offs = jnp.arange(B, dtype=edge_index.dtype) * N
    src_glob = (edge_index[0][None, :] + offs[:, None]).reshape(-1)
    dst_loc = jnp.concatenate([edge_index[1], edge_index[1]])
    x = data[:, :N * F].reshape(B * N, F)

    zeros_64 = jnp.zeros((ROWS_T, F // 2), jnp.float32)
    zeros_h = jnp.zeros((ROWS_T, H), jnp.float32)

    a_lo, a_hi = _segsum_2x64(x[:, :F // 2], x[:, F // 2:],
                              src_glob, dst_loc, zeros_64)
    agg = jnp.concatenate([a_lo, a_hi], axis=1)
    h = jax.nn.relu(agg @ W_rel.T + b_rel + x @ W_root.T)
    agg2 = _segsum(h, src_glob, dst_loc, zeros_h)

    score = (agg2 @ Ws_rel.T + bs_rel + h @ Ws_root.T).reshape(B, N)
    score = jnp.tanh(score)
    u = lax.bitcast_convert_type(score.reshape(B * N), jnp.int32)
    keys = ~(u ^ ((u >> 31) | jnp.int32(-2147483648)))
    vals = jnp.arange(B * N, dtype=jnp.int32)

    hp = _topk_pool(keys, vals, h)

    w3p = jnp.concatenate([W3, jnp.zeros((7, 64), jnp.float32)], axis=0)
    out = _fc_head(hp.reshape(B, K * H), W1, b1.reshape(1, 256),
                   W2, b2.reshape(1, 64), w3p, b3.reshape(1, 1))
    return out[:, :1]


# speedup vs baseline: 2.4514x; 1.0004x over previous
import functools

import jax
import jax.numpy as jnp
from jax import lax
from jax.experimental import pallas as pl
from jax.experimental.pallas import tpu as pltpu
from jax.experimental.pallas import tpu_sc as plsc

N = 10000
F = 128
H = 16
B = 2
E = 160000
K = 4000

NT = 16
ROWS_T = 624

NV = N // 16
TR = 256


def _sc_mesh():
    return plsc.VectorSubcoreMesh(core_axis_name="c", subcore_axis_name="s")


def _segsum(table, src_glob, dst_loc, zeros_tile):
    W = table.shape[1]
    CHUNK = 2000
    NCH = E // CHUNK
    MAXJ = NCH // NT
    params = (None if W == F
              else pltpu.CompilerParams(use_tc_tiling_on_sc=False))

    @functools.partial(
        pl.kernel,
        out_type=jax.ShapeDtypeStruct((B * N, W), jnp.float32),
        mesh=_sc_mesh(),
        scratch_types=[
            pltpu.VMEM((CHUNK,), jnp.int32),
            pltpu.VMEM((CHUNK,), jnp.int32),
            pltpu.VMEM((CHUNK, W), jnp.float32),
            pltpu.SemaphoreType.DMA,
            pltpu.VMEM_SHARED((N, W), jnp.float32),
        ],
        compiler_params=params,
    )
    def seg(table_hbm, src_hbm, dst_hbm, zeros_hbm, out_hbm,
            src_v, dst_v, rows_v, sem, acc):
        cid = lax.axis_index("c")
        sid = lax.axis_index("s")
        pltpu.sync_copy(zeros_hbm, acc.at[pl.ds(sid * ROWS_T, ROWS_T)])

        @pl.when(sid == NT - 1)
        def _():
            pltpu.sync_copy(zeros_hbm.at[pl.ds(0, 16)],
                            acc.at[pl.ds(NT * ROWS_T, 16)])

        plsc.subcore_barrier()
        ebase = cid * E

        for r in range(MAXJ):
            j = r * NT + sid
            base = ebase + j * CHUNK
            pltpu.sync_copy(src_hbm.at[pl.ds(base, CHUNK)], src_v)
            pltpu.sync_copy(dst_hbm.at[pl.ds(base, CHUNK)], dst_v)
            pltpu.async_copy(table_hbm.at[src_v], rows_v, sem).wait()
            for k in range(NT):
                @pl.when(sid == k)
                def _():
                    pltpu.sync_copy(rows_v, acc.at[dst_v], add=True)

                plsc.subcore_barrier()
        pltpu.sync_copy(
            acc.at[pl.ds(sid * ROWS_T, ROWS_T)],
            out_hbm.at[pl.ds(cid * N + sid * ROWS_T, ROWS_T)],
        )

        @pl.when(sid == NT - 1)
        def _():
            pltpu.sync_copy(
                acc.at[pl.ds(NT * ROWS_T, 16)],
                out_hbm.at[pl.ds(cid * N + NT * ROWS_T, 16)],
            )

    return seg(table, src_glob, dst_loc, zeros_tile)


def _topk_pool(keys, vals, h):
    @functools.partial(
        pl.kernel,
        out_type=jax.ShapeDtypeStruct((B, K, H), jnp.float32),
        mesh=_sc_mesh(),
        scratch_types=[
            pltpu.VMEM((N,), jnp.int32),
            pltpu.VMEM((N,), jnp.int32),
            pltpu.VMEM((N,), jnp.int32),
            pltpu.VMEM((N,), jnp.int32),
            pltpu.VMEM((256,), jnp.int32),
            pltpu.VMEM((256,), jnp.int32),
            pltpu.VMEM((TR,), jnp.float32),
            pltpu.VMEM((TR, H), jnp.float32),
            pltpu.SemaphoreType.DMA,
        ],
        compiler_params=pltpu.CompilerParams(needs_layout_passes=False,
                                             use_tc_tiling_on_sc=False),
    )
    def srt(keys_hbm, vals_hbm, h_hbm, hp_hbm,
            ka, va, kb, vb, hist, offs, sval_v, prow_v, sem):
        cid = lax.axis_index("c")
        sid = lax.axis_index("s")

        pltpu.sync_copy(keys_hbm.at[pl.ds(cid * N, N)], ka)
        pltpu.sync_copy(vals_hbm.at[pl.ds(cid * N, N)], va)
        for p in range(4):
            ink, inv = (ka, va) if p % 2 == 0 else (kb, vb)
            outk, outv = (kb, vb) if p % 2 == 0 else (ka, va)
            shift = 8 * p
            for t in range(16):
                hist[pl.ds(t * 16, 16)] = jnp.zeros((16,), jnp.int32)

            def hist_body(v, carry, ink=ink, shift=shift):
                kv = ink[pl.ds(v * 16, 16)]
                d = lax.shift_right_logical(kv, shift) & 0xFF
                c, last = plsc.scan_count(d)
                plsc.addupdate_scatter(hist, [d], c, mask=last)
                return carry

            lax.fori_loop(0, NV, hist_body, 0)
            carry = jnp.int32(0)
            for t in range(16):
                hv = hist[pl.ds(t * 16, 16)]
                cs = plsc.cumsum(hv)
                offs[pl.ds(t * 16, 16)] = cs - hv + carry
                carry = carry + cs[15]

            def perm_body(v, carry, ink=ink, inv=inv, outk=outk,
                          outv=outv, shift=shift):
                kv = ink[pl.ds(v * 16, 16)]
                vv = inv[pl.ds(v * 16, 16)]
                d = lax.shift_right_logical(kv, shift) & 0xFF
                c, last = plsc.scan_count(d)
                base = plsc.load_gather(offs, [d])
                pos = base + c - 1
                plsc.store_scatter(outk, [pos], kv)
                plsc.store_scatter(outv, [pos], vv)
                plsc.addupdate_scatter(offs, [d], c, mask=last)
                return carry

            lax.fori_loop(0, NV, perm_body, 0)

        rbase = sid * TR

        def score_body(t, carry):
            kk = ka[pl.ds(rbase + t * 16, 16)]
            u = kk ^ (jnp.int32(0x7FFFFFFF) & ~(kk >> 31))
            sval_v[pl.ds(t * 16, 16)] = plsc.bitcast(u, jnp.float32)
            return carry

        lax.fori_loop(0, TR // 16, score_body, 0)

        pltpu.async_copy(h_hbm.at[va.at[pl.ds(rbase, TR)]], prow_v, sem).wait()

        def scale_body(g, carry):
            s16 = sval_v[pl.ds(g * 16, 16)]
            ridx = g * 16 + lax.iota(jnp.int32, 16)
            for c in range(H):
                cidx = jnp.full((16,), c, jnp.int32)
                rv = plsc.load_gather(prow_v, [ridx, cidx])
                plsc.store_scatter(prow_v, [ridx, cidx], rv * s16)
            return carry

        lax.fori_loop(0, TR // 16, scale_body, 0)

        @pl.when(sid < NT - 1)
        def _out_full():
            pltpu.sync_copy(prow_v, hp_hbm.at[cid, pl.ds(rbase, TR)])

        @pl.when(sid == NT - 1)
        def _out_tail():
            pltpu.sync_copy(prow_v.at[pl.ds(0, K - (NT - 1) * TR)],
                            hp_hbm.at[cid, pl.ds(rbase, K - (NT - 1) * TR)])

    return srt(keys, vals, h)


def _segsum_2x64(xlo, xhi, src_glob, dst_loc, zeros_tile):
    W = F // 2
    CHUNK = 1000
    NCH = E // CHUNK
    MAXJ = NCH // NT

    @functools.partial(
        pl.kernel,
        out_type=(jax.ShapeDtypeStruct((B * N, W), jnp.float32),
                  jax.ShapeDtypeStruct((B * N, W), jnp.float32)),
        mesh=_sc_mesh(),
        scratch_types=[
            pltpu.VMEM((CHUNK,), jnp.int32),
            pltpu.VMEM((CHUNK,), jnp.int32),
            pltpu.VMEM((CHUNK, W), jnp.float32),
            pltpu.SemaphoreType.DMA,
            pltpu.VMEM_SHARED((N, W), jnp.float32),
        ],
        compiler_params=pltpu.CompilerParams(use_tc_tiling_on_sc=False),
    )
    def seg(xlo_hbm, xhi_hbm, src_hbm, dst_hbm, zeros_hbm,
            outlo_hbm, outhi_hbm, src_v, dst_v, rows_v, sem, acc):
        cid = lax.axis_index("c")
        sid = lax.axis_index("s")
        ebase = cid * E
        for table_hbm, out_hbm in ((xlo_hbm, outlo_hbm), (xhi_hbm, outhi_hbm)):
            pltpu.sync_copy(zeros_hbm, acc.at[pl.ds(sid * ROWS_T, ROWS_T)])

            @pl.when(sid == NT - 1)
            def _():
                pltpu.sync_copy(zeros_hbm.at[pl.ds(0, 16)],
                                acc.at[pl.ds(NT * ROWS_T, 16)])

            plsc.subcore_barrier()
            for r in range(MAXJ):
                j = r * NT + sid
                base = ebase + j * CHUNK
                pltpu.sync_copy(src_hbm.at[pl.ds(base, CHUNK)], src_v)
                pltpu.sync_copy(dst_hbm.at[pl.ds(base, CHUNK)], dst_v)
                pltpu.async_copy(table_hbm.at[src_v], rows_v, sem).wait()
                for k in range(NT):
                    @pl.when(sid == k)
                    def _():
                        pltpu.sync_copy(rows_v, acc.at[dst_v], add=True)

                    plsc.subcore_barrier()

            pltpu.sync_copy(
                acc.at[pl.ds(sid * ROWS_T, ROWS_T)],
                out_hbm.at[pl.ds(cid * N + sid * ROWS_T, ROWS_T)],
            )

            @pl.when(sid == NT - 1)
            def _():
                pltpu.sync_copy(
                    acc.at[pl.ds(NT * ROWS_T, 16)],
                    out_hbm.at[pl.ds(cid * N + NT * ROWS_T, 16)],
                )

    return seg(xlo, xhi, src_glob, dst_loc, zeros_tile)


def _fc_head(xf, w1, b1_2, w2, b2_2, w3, b3_2):
    CK = 6400
    steps = (K * H) // CK

    def body(xf_ref, w1_ref, b1_ref, w2_ref, b2_ref, w3_ref, b3_ref,
             out_ref, acc):
        i = pl.program_id(0)

        @pl.when(i == 0)
        def _():
            acc[...] = jnp.zeros_like(acc)

        acc[...] += lax.dot_general(xf_ref[...], w1_ref[...],
                                    (((1,), (1,)), ((), ())),
                                    preferred_element_type=jnp.float32)

        @pl.when(i == steps - 1)
        def _():
            y1 = jnp.maximum(acc[...] + b1_ref[...], 0.0)
            y2 = lax.dot_general(y1, w2_ref[...], (((1,), (1,)), ((), ())),
                                 preferred_element_type=jnp.float32)
            y2 = jnp.maximum(y2 + b2_ref[...], 0.0)
            y3 = lax.dot_general(y2, w3_ref[...], (((1,), (1,)), ((), ())),
                                 preferred_element_type=jnp.float32)
            out_ref[...] = y3 + b3_ref[0, 0]

    return pl.pallas_call(
        body,
        grid=(steps,),
        in_specs=[
            pl.BlockSpec((B, CK), lambda i: (0, i)),
            pl.BlockSpec((256, CK), lambda i: (0, i)),
            pl.BlockSpec((1, 256), lambda i: (0, 0)),
            pl.BlockSpec((64, 256), lambda i: (0, 0)),
            pl.BlockSpec((1, 64), lambda i: (0, 0)),
            pl.BlockSpec((8, 64), lambda i: (0, 0)),
            pl.BlockSpec((1, 1), lambda i: (0, 0)),
        ],
        out_specs=pl.BlockSpec((B, 8), lambda i: (0, 0)),
        out_shape=jax.ShapeDtypeStruct((B, 8), jnp.float32),
        scratch_shapes=[pltpu.VMEM((B, 256), jnp.float32)],
    )(xf, w1, b1_2, w2, b2_2, w3, b3_2)


def kernel(data, edge_index, W_rel, b_rel, W_root, Ws_rel, bs_rel, Ws_root,
           W1, b1, W2, b2, W3, b3):
    offs = jnp.arange(B, dtype=edge_index.dtype) * N
    src_glob = (edge_index[0][None, :] + offs[:, None]).reshape(-1)
    dst_loc = jnp.concatenate([edge_index[1], edge_index[1]])
    x = data[:, :N * F].reshape(B * N, F)

    zeros_64 = jnp.zeros((ROWS_T, F // 2), jnp.float32)
    zeros_h = jnp.zeros((ROWS_T, H), jnp.float32)

    a_lo, a_hi = _segsum_2x64(x[:, :F // 2], x[:, F // 2:],
                              src_glob, dst_loc, zeros_64)
    agg = jnp.concatenate([a_lo, a_hi], axis=1)
    h = jax.nn.relu(agg @ W_rel.T + b_rel + x @ W_root.T)
    agg2 = _segsum(h, src_glob, dst_loc, zeros_h)

    score = (agg2 @ Ws_rel.T + bs_rel + h @ Ws_root.T).reshape(B, N)
    score = jnp.tanh(score)
    u = lax.bitcast_convert_type(score.reshape(B * N), jnp.int32)
    keys = ~(u ^ ((u >> 31) | jnp.int32(-2147483648)))
    vals = jnp.arange(B * N, dtype=jnp.int32)

    hp = _topk_pool(keys, vals, h)

    w3p = jnp.concatenate([W3, jnp.zeros((7, 64), jnp.float32)], axis=0)
    out = _fc_head(hp.reshape(B, K * H), W1, b1.reshape(1, 256),
                   W2, b2.reshape(1, 64), w3p, b3.reshape(1, 1))
    return out[:, :1]
